# baseline (speedup 1.0000x reference)
import functools

import jax
import jax.numpy as jnp
from jax import lax
from jax.experimental import pallas as pl
from jax.experimental.pallas import tpu as pltpu
from jax.experimental.pallas import tpu_sc as plsc

NC = 2
NS = 16
L = 16


def _complex_score_kernel(B, D, CB):
    NW = NC * NS
    BPW = B // NW
    NCHUNK = BPW // CB
    W = 2 * D

    mesh = plsc.VectorSubcoreMesh(core_axis_name="c", subcore_axis_name="s")

    @functools.partial(
        pl.kernel,
        out_type=jax.ShapeDtypeStruct((B,), jnp.float32),
        mesh=mesh,
        compiler_params=pltpu.CompilerParams(
            needs_layout_passes=False, use_tc_tiling_on_sc=True),
        scratch_types=[
            pltpu.VMEM((BPW,), jnp.int32),
            pltpu.VMEM((BPW,), jnp.int32),
            pltpu.VMEM((BPW,), jnp.int32),
            pltpu.VMEM((BPW,), jnp.int32),
            pltpu.VMEM((BPW,), jnp.int32),
            pltpu.VMEM((BPW,), jnp.int32),
            pltpu.VMEM((CB, 128), jnp.float32),
            pltpu.VMEM((CB, 128), jnp.float32),
            pltpu.VMEM((CB, 128), jnp.float32),
            pltpu.VMEM((CB, 128), jnp.float32),
            pltpu.VMEM((CB, 128), jnp.float32),
            pltpu.VMEM((CB, 128), jnp.float32),
            pltpu.VMEM((BPW,), jnp.float32),
            pltpu.SemaphoreType.DMA,
        ],
    )
    def k(hidx_hbm, ridx_hbm, tidx_hbm, nre_hbm, nim_hbm, rre_hbm, rim_hbm,
          out_hbm, hidx_v, ridx_v, tidx_v, hrow_v, rrow_v, trow_v,
          hr_v, hi_v, rr_v, ri_v, tr_v, ti_v, scores_v, sem):
        wid = lax.axis_index("s") * NC + lax.axis_index("c")
        base = wid * BPW
        pltpu.sync_copy(hidx_hbm.at[pl.ds(base, BPW)], hidx_v)
        pltpu.sync_copy(ridx_hbm.at[pl.ds(base, BPW)], ridx_v)
        pltpu.sync_copy(tidx_hbm.at[pl.ds(base, BPW)], tidx_v)

        def rowify(g, carry):
            sl = pl.ds(pl.multiple_of(g * L, L), L)
            hrow_v[sl] = lax.shift_right_logical(hidx_v[sl], 1)
            rrow_v[sl] = lax.shift_right_logical(ridx_v[sl], 1)
            trow_v[sl] = lax.shift_right_logical(tidx_v[sl], 1)
            return carry

        lax.fori_loop(0, BPW // L, rowify, 0)

        iota = lax.iota(jnp.int32, L)

        def chunk(c, carry):
            cofs = pl.multiple_of(c * CB, CB)
            hslice = hrow_v.at[pl.ds(cofs, CB)]
            rslice = rrow_v.at[pl.ds(cofs, CB)]
            tslice = trow_v.at[pl.ds(cofs, CB)]
            cps = [
                pltpu.async_copy(nre_hbm.at[hslice], hr_v, sem),
                pltpu.async_copy(nim_hbm.at[hslice], hi_v, sem),
                pltpu.async_copy(rre_hbm.at[rslice], rr_v, sem),
                pltpu.async_copy(rim_hbm.at[rslice], ri_v, sem),
                pltpu.async_copy(nre_hbm.at[tslice], tr_v, sem),
                pltpu.async_copy(nim_hbm.at[tslice], ti_v, sem),
            ]
            for cp in cps:
                cp.wait()

            def group(g, carry2):
                gofs = pl.multiple_of(g * L, L)
                sl = pl.ds(cofs + gofs, L)
                rows = iota + gofs
                hh = lax.shift_left(jnp.bitwise_and(hidx_v[sl], 1), 6)
                rh = lax.shift_left(jnp.bitwise_and(ridx_v[sl], 1), 6)
                th = lax.shift_left(jnp.bitwise_and(tidx_v[sl], 1), 6)
                acc = jnp.zeros((L,), jnp.float32)
                for h in range(D):
                    hr = plsc.load_gather(hr_v, [rows, hh + h])
                    hi = plsc.load_gather(hi_v, [rows, hh + h])
                    rr = plsc.load_gather(rr_v, [rows, rh + h])
                    ri = plsc.load_gather(ri_v, [rows, rh + h])
                    tr = plsc.load_gather(tr_v, [rows, th + h])
                    ti = plsc.load_gather(ti_v, [rows, th + h])
                    acc = acc + rr * (hr * tr + hi * ti) \
                              + ri * (hr * ti - hi * tr)
                scores_v[sl] = acc
                return carry2

            lax.fori_loop(0, CB // L, group, 0)
            return carry

        lax.fori_loop(0, NCHUNK, chunk, 0)
        pltpu.sync_copy(scores_v, out_hbm.at[pl.ds(base, BPW)])

    return k


def kernel(head_index, rel_type, tail_index, node_emb, node_emb_im,
           rel_emb, rel_emb_im):
    B = head_index.shape[0]
    V, D = node_emb.shape
    R = rel_emb.shape[0]
    W = 2 * D
    k = _complex_score_kernel(B, D, CB=128)
    nre2 = node_emb.reshape(V * D // W, W)
    nim2 = node_emb_im.reshape(V * D // W, W)
    rre2 = rel_emb.reshape(R * D // W, W)
    rim2 = rel_emb_im.reshape(R * D // W, W)
    return k(head_index.astype(jnp.int32), rel_type.astype(jnp.int32),
             tail_index.astype(jnp.int32), nre2, nim2, rre2, rim2)
